# baseline (speedup 1.0000x reference)
import functools

import jax
import jax.numpy as jnp
from jax import lax
from jax.experimental import pallas as pl
from jax.experimental.pallas import tpu as pltpu
from jax.experimental.pallas import tpu_sc as plsc

B, D = 16384, 32
K, CF = 64, 8
S, NREQ = 4, 8
CD_W = 16

BS = 2048
BT = B - BS

_NC, _NS, _L = 2, 16, 16
_NW = _NC * _NS
_RPW = BS // _NW
_G2 = _RPW // _L


def _cd_table(rsw, coefs, bias2d):
    a = jnp.abs(rsw)
    wn = a / jnp.maximum(jnp.sum(a, axis=1, keepdims=True), 1e-12)
    cmat = lax.dot_general(wn, coefs, (((1,), (0,)), ((), ())),
                           preferred_element_type=jnp.float32)
    dvec = lax.dot_general(wn, bias2d, (((1,), (1,)), ((), ())),
                           preferred_element_type=jnp.float32)
    return jnp.concatenate(
        [cmat, dvec, jnp.zeros((K, CD_W - NREQ - 1), jnp.float32)], axis=1)


def _tc_body(x_ref, cf_ref, cent_ref, rsw_ref, coef_ref, bias_ref, out_ref):
    x = x_ref[...]
    cf = cf_ref[...]
    cent = cent_ref[...]

    sel = (lax.broadcasted_iota(jnp.int32, (D, CF), 0) == cf
           ).astype(jnp.float32)
    G = lax.dot_general(sel, -2.0 * cent, (((1,), (1,)), ((), ())),
                        preferred_element_type=jnp.float32)
    c0 = lax.dot_general(jnp.ones((1, CF), jnp.float32), cent * cent,
                         (((1,), (1,)), ((), ())),
                         preferred_element_type=jnp.float32)
    scores = lax.dot_general(x, G, (((1,), (0,)), ((), ())),
                             preferred_element_type=jnp.float32) + c0

    ib = lax.bitcast_convert_type(scores, jnp.int32)
    key = ib ^ lax.shift_right_logical(
        lax.shift_right_arithmetic(ib, 31), 1)
    iota_k = lax.broadcasted_iota(jnp.int32, (BT, K), 1)
    ckey = (key & jnp.int32(-64)) | iota_k
    mkey = jnp.min(ckey, axis=1, keepdims=True)
    onehot = (ckey == mkey).astype(jnp.float32)

    cd = _cd_table(rsw_ref[...], coef_ref[...], bias_ref[...])
    cdg = lax.dot_general(onehot, cd, (((1,), (0,)), ((), ())),
                          preferred_element_type=jnp.float32)
    xmod = jnp.concatenate(
        [x[:, :NREQ], jnp.ones((BT, 1), jnp.float32),
         jnp.zeros((BT, CD_W - NREQ - 1), jnp.float32)], axis=1)
    out_ref[...] = jnp.sum(cdg * xmod, axis=1, keepdims=True)


def _tc_part(x_top, cf2d, cent, rsw, coefs, bias2d):
    return pl.pallas_call(
        _tc_body,
        out_shape=jax.ShapeDtypeStruct((BT, 1), jnp.float32),
    )(x_top, cf2d, cent, rsw, coefs, bias2d)


def _sc_body(x_hbm, cf_hbm, cent_hbm, rsw_hbm, coef_hbm, bias_hbm, out_hbm,
             x_v, cf_v, cent_v, rsw_v, coef_v, bias_v, cd_v, out_v):
    wid = lax.axis_index("s") * _NC + lax.axis_index("c")
    row0 = BT + wid * _RPW
    pltpu.sync_copy(x_hbm.at[pl.ds(row0 * D, _RPW * D)], x_v)
    pltpu.sync_copy(cf_hbm, cf_v)
    pltpu.sync_copy(cent_hbm, cent_v)
    pltpu.sync_copy(rsw_hbm, rsw_v)
    pltpu.sync_copy(coef_hbm, coef_v)
    pltpu.sync_copy(bias_hbm, bias_v)

    lanes = lax.iota(jnp.int32, _L)

    def splat(ref, idx):
        return plsc.load_gather(ref, [jnp.full((_L,), idx, jnp.int32)])

    for kg in range(K // _L):
        k16 = kg * _L + lanes
        a = [jnp.abs(plsc.load_gather(rsw_v, [k16 * S + s]))
             for s in range(S)]
        tot = jnp.maximum(a[0] + a[1] + a[2] + a[3], 1e-12)
        wn = [ai / tot for ai in a]
        for j in range(NREQ):
            col = wn[0] * splat(coef_v, 0 * NREQ + j)
            for s in range(1, S):
                col = col + wn[s] * splat(coef_v, s * NREQ + j)
            plsc.store_scatter(cd_v, [k16 * CD_W + j], col)
        dcol = wn[0] * splat(bias_v, 0)
        for s in range(1, S):
            dcol = dcol + wn[s] * splat(bias_v, s)
        plsc.store_scatter(cd_v, [k16 * CD_W + NREQ], dcol)

    cfs = [splat(cf_v, j) for j in range(CF)]

    def group(g, carry):
        rb = (g * _L + lanes) * D
        xc = [plsc.load_gather(x_v, [rb + cfs[j]]) for j in range(CF)]
        minv = jnp.full((_L,), jnp.inf, jnp.float32)
        argv = jnp.zeros((_L,), jnp.int32)
        for k in range(K):
            t = xc[0] - splat(cent_v, k * CF + 0)
            ds = t * t
            for j in range(1, CF):
                t = xc[j] - splat(cent_v, k * CF + j)
                ds = ds + t * t
            m = ds < minv
            minv = jnp.where(m, ds, minv)
            argv = jnp.where(m, jnp.full((_L,), k, jnp.int32), argv)
        cdbase = argv * CD_W
        acc = plsc.load_gather(cd_v, [cdbase + NREQ])
        for j in range(NREQ):
            xj = plsc.load_gather(x_v, [rb + j])
            cj = plsc.load_gather(cd_v, [cdbase + j])
            acc = acc + xj * cj
        out_v[pl.ds(g * _L, _L)] = acc
        return carry

    lax.fori_loop(0, _G2, group, 0)
    pltpu.sync_copy(out_v, out_hbm.at[pl.ds(wid * _RPW, _RPW)])


@functools.cache
def _sc_part():
    return pl.kernel(
        _sc_body,
        out_type=jax.ShapeDtypeStruct((BS,), jnp.float32),
        mesh=plsc.VectorSubcoreMesh(core_axis_name="c", subcore_axis_name="s"),
        compiler_params=pltpu.CompilerParams(needs_layout_passes=False),
        scratch_types=[
            pltpu.VMEM((_RPW * D,), jnp.float32),
            pltpu.VMEM((CF,), jnp.int32),
            pltpu.VMEM((K * CF,), jnp.float32),
            pltpu.VMEM((K * S,), jnp.float32),
            pltpu.VMEM((S * NREQ,), jnp.float32),
            pltpu.VMEM((S,), jnp.float32),
            pltpu.VMEM((K * CD_W,), jnp.float32),
            pltpu.VMEM((_RPW,), jnp.float32),
        ],
    )


def kernel(x, clustering_features, centroids, running_sn_weight, sn_coefs,
           sn_bias):
    cf = clustering_features.astype(jnp.int32)
    out_sc = _sc_part()(
        x.reshape(B * D), cf, centroids.reshape(K * CF),
        running_sn_weight.reshape(K * S), sn_coefs.reshape(S * NREQ), sn_bias)
    out_tc = _tc_part(x[:BT], cf.reshape(1, CF), centroids,
                      running_sn_weight, sn_coefs, sn_bias.reshape(1, S))
    return jnp.concatenate([out_tc, out_sc.reshape(BS, 1)], axis=0)
